# Initial kernel scaffold; baseline (speedup 1.0000x reference)
import jax
import jax.numpy as jnp
from jax.experimental import pallas as pl


def kernel(x, edge_index, edge_attr, W1, b1, W2, b2):
    raise NotImplementedError("write your pallas kernel here")


# speedup vs baseline: 2.6791x; 2.6791x over previous
import functools

import jax
import jax.numpy as jnp
from jax import lax
from jax.experimental import pallas as pl
from jax.experimental.pallas import tpu as pltpu
from jax.experimental.pallas import tpu_sc as plsc

N_NODES = 10000
N_EDGES = 320000
D = 128
DH = 64
D_EDGE = 16

NT = 16
C = 128
EPT = 20480
E_PAD = NT * EPT
N_CHUNKS = EPT // C
ACC_ROWS = 10240
ROWS_PT = ACC_ROWS // NT
N_PAD_ROWS = ACC_ROWS - N_NODES


def _tables_body(x_ref, wa_ref, wb_ref, xa_ref, xb_ref):
    x = x_ref[...]
    xa = jnp.dot(x, wa_ref[...], preferred_element_type=jnp.float32)
    xb = jnp.dot(x, wb_ref[...], preferred_element_type=jnp.float32)
    xa_ref[0] = xa[:, :DH]
    xa_ref[1] = xa[:, DH:]
    xb_ref[0] = xb[:, :DH]
    xb_ref[1] = xb[:, DH:]


def _make_tables(x_pad, w1at, w1bt):
    return pl.pallas_call(
        _tables_body,
        out_shape=(
            jax.ShapeDtypeStruct((2, ACC_ROWS, DH), jnp.float32),
            jax.ShapeDtypeStruct((2, ACC_ROWS, DH), jnp.float32),
        ),
    )(x_pad, w1at, w1bt)


_EB = 4096


def _ea_body(attr_ref, wc_ref, b1_ref, ea_ref):
    v = (
        jnp.dot(attr_ref[...], wc_ref[...], preferred_element_type=jnp.float32)
        + b1_ref[...]
    )
    ea_ref[0] = v[:, :DH]
    ea_ref[1] = v[:, DH:]


def _make_ea(attr_pad, w1ct, b1r):
    grid = (E_PAD // _EB,)
    return pl.pallas_call(
        _ea_body,
        grid=grid,
        in_specs=[
            pl.BlockSpec((_EB, D_EDGE), lambda i: (i, 0)),
            pl.BlockSpec((D_EDGE, D), lambda i: (0, 0)),
            pl.BlockSpec((1, D), lambda i: (0, 0)),
        ],
        out_specs=pl.BlockSpec((2, _EB, DH), lambda i: (0, i, 0)),
        out_shape=jax.ShapeDtypeStruct((2, E_PAD, DH), jnp.float32),
    )(attr_pad, w1ct, b1r)


def _sc_body(
    xa_hbm, xb_hbm, ea_hbm, src_hbm, dst_hbm,
    acc_out, cnt_out,
    idx_src, idx_dst, idx_srca, idx_dsta,
    rows_a, rows_b, ea_buf,
    ones_buf, zero_buf, zero_cnt,
    acc_sh, cnt_sh,
    sem_a, sem_b, sem_e,
):
    cid = lax.axis_index("c")
    sid = lax.axis_index("s")
    row_off = cid * ACC_ROWS
    ea_off = cid * E_PAD

    def fill_rows(i, _):
        for j in range(DH // 16):
            zero_buf[i, pl.ds(j * 16, 16)] = jnp.zeros((16,), jnp.float32)
        ones_buf[i, pl.ds(0, 16)] = jnp.ones((16,), jnp.float32)
        return 0

    lax.fori_loop(0, C, fill_rows, 0)

    def fill_zc(i, _):
        zero_cnt[i, pl.ds(0, 16)] = jnp.zeros((16,), jnp.float32)
        return 0

    lax.fori_loop(0, ROWS_PT, fill_zc, 0)

    for k in range(ROWS_PT // C):
        pltpu.sync_copy(zero_buf, acc_sh.at[pl.ds(sid * ROWS_PT + k * C, C)])
    pltpu.sync_copy(zero_cnt, cnt_sh.at[pl.ds(sid * ROWS_PT, ROWS_PT)])
    plsc.subcore_barrier()

    ebase = sid * EPT

    def chunk(k, _):
        base = ebase + k * C
        pltpu.sync_copy(src_hbm.at[pl.ds(base, C)], idx_src)
        pltpu.sync_copy(dst_hbm.at[pl.ds(base, C)], idx_dst)
        for j in range(C // 16):
            s = pl.ds(j * 16, 16)
            idx_srca[s] = idx_src[s] + row_off
            idx_dsta[s] = idx_dst[s] + row_off
        ca = pltpu.async_copy(xa_hbm.at[idx_dsta], rows_a, sem_a)
        cb = pltpu.async_copy(xb_hbm.at[idx_srca], rows_b, sem_b)
        ce = pltpu.async_copy(ea_hbm.at[pl.ds(ea_off + base, C)], ea_buf, sem_e)
        ca.wait()
        cb.wait()
        ce.wait()

        def row(i, _):
            for j in range(DH // 16):
                s = pl.ds(j * 16, 16)
                v = rows_a[i, s] + rows_b[i, s] + ea_buf[i, s]
                rows_a[i, s] = jnp.maximum(v, 0.0)
            return 0

        lax.fori_loop(0, C, row, 0)

        pltpu.sync_copy(rows_a, acc_sh.at[idx_dst], add=True)

        @pl.when(cid == 0)
        def _():
            pltpu.sync_copy(ones_buf, cnt_sh.at[idx_dst], add=True)

        return 0

    lax.fori_loop(0, N_CHUNKS, chunk, 0)
    plsc.subcore_barrier()

    r0 = sid * ROWS_PT
    pltpu.sync_copy(acc_sh.at[pl.ds(r0, ROWS_PT)], acc_out.at[cid, pl.ds(r0, ROWS_PT)])

    @pl.when(cid == 0)
    def _():
        pltpu.sync_copy(cnt_sh.at[pl.ds(r0, ROWS_PT)], cnt_out.at[pl.ds(r0, ROWS_PT)])


def _make_sc(xa, xb, ea, src_p, dst_p):
    mesh = plsc.VectorSubcoreMesh(core_axis_name="c", subcore_axis_name="s")
    f = functools.partial(
        pl.kernel,
        compiler_params=pltpu.CompilerParams(use_tc_tiling_on_sc=False),
        out_type=(
            jax.ShapeDtypeStruct((2, ACC_ROWS, DH), jnp.float32),
            jax.ShapeDtypeStruct((ACC_ROWS, 16), jnp.float32),
        ),
        mesh=mesh,
        scratch_types=[
            pltpu.VMEM((C,), jnp.int32),
            pltpu.VMEM((C,), jnp.int32),
            pltpu.VMEM((C,), jnp.int32),
            pltpu.VMEM((C,), jnp.int32),
            pltpu.VMEM((C, DH), jnp.float32),
            pltpu.VMEM((C, DH), jnp.float32),
            pltpu.VMEM((C, DH), jnp.float32),
            pltpu.VMEM((C, 16), jnp.float32),
            pltpu.VMEM((C, DH), jnp.float32),
            pltpu.VMEM((ROWS_PT, 16), jnp.float32),
            pltpu.VMEM_SHARED((ACC_ROWS, DH), jnp.float32),
            pltpu.VMEM_SHARED((ACC_ROWS, 16), jnp.float32),
            pltpu.SemaphoreType.DMA,
            pltpu.SemaphoreType.DMA,
            pltpu.SemaphoreType.DMA,
        ],
    )(_sc_body)
    return f(xa, xb, ea, src_p, dst_p)


def _finish_body(acc_ref, cnt_ref, w2t_ref, b2_ref, out_ref):
    s = jnp.concatenate(
        [acc_ref[0, :N_NODES, :], acc_ref[1, :N_NODES, :]], axis=1
    )
    c = jnp.maximum(cnt_ref[:N_NODES, 0:1], 1.0)
    out_ref[...] = (
        jnp.dot(s / c, w2t_ref[...], preferred_element_type=jnp.float32)
        + b2_ref[...]
    )


def _make_finish(acc, cnt, w2t, b2r):
    return pl.pallas_call(
        _finish_body,
        out_shape=jax.ShapeDtypeStruct((N_NODES, D), jnp.float32),
    )(acc, cnt, w2t, b2r)


@jax.jit
def kernel(x, edge_index, edge_attr, W1, b1, W2, b2):
    src = edge_index[0].astype(jnp.int32)
    dst = edge_index[1].astype(jnp.int32)

    n_pad = E_PAD - N_EDGES
    pad_idx = (N_NODES + jnp.arange(n_pad, dtype=jnp.int32) % N_PAD_ROWS)
    src_p = jnp.concatenate([src, pad_idx])
    dst_p = jnp.concatenate([dst, pad_idx])
    attr_pad = jnp.concatenate(
        [edge_attr, jnp.zeros((n_pad, D_EDGE), jnp.float32)]
    )
    x_pad = jnp.concatenate(
        [x, jnp.zeros((ACC_ROWS - N_NODES, x.shape[1]), jnp.float32)]
    )

    w1at = W1[:, :D].T
    w1bt = W1[:, D:2 * D].T
    w1ct = W1[:, 2 * D:].T
    b1r = b1.reshape(1, D)
    b2r = b2.reshape(1, D)

    xa, xb = _make_tables(x_pad, w1at, w1bt)
    ea = _make_ea(attr_pad, w1ct, b1r)
    xa2 = xa.reshape(2 * ACC_ROWS, DH)
    xb2 = xb.reshape(2 * ACC_ROWS, DH)
    ea2 = ea.reshape(2 * E_PAD, DH)
    acc, cnt = _make_sc(xa2, xb2, ea2, src_p, dst_p)
    return _make_finish(acc, cnt, W2.T, b2r)


# speedup vs baseline: 4.6501x; 1.7357x over previous
import functools

import jax
import jax.numpy as jnp
from jax import lax
from jax.experimental import pallas as pl
from jax.experimental.pallas import tpu as pltpu
from jax.experimental.pallas import tpu_sc as plsc

N_NODES = 10000
N_EDGES = 320000
D = 128
DH = 64
D_EDGE = 16

NT = 16
C = 128
EPT = 20480
E_PAD = NT * EPT
N_CHUNKS = EPT // C
ACC_ROWS = 10240
ROWS_PT = ACC_ROWS // NT
N_PAD_ROWS = ACC_ROWS - N_NODES


def _tables_body(x_ref, wa_ref, wb_ref, xa_ref, xb_ref):
    x = x_ref[...]
    xa = jnp.dot(x, wa_ref[...], preferred_element_type=jnp.float32)
    xb = jnp.dot(x, wb_ref[...], preferred_element_type=jnp.float32)
    xa_ref[pl.ds(0, ACC_ROWS), :] = xa[:, :DH]
    xa_ref[pl.ds(ACC_ROWS, ACC_ROWS), :] = xa[:, DH:]
    xb_ref[pl.ds(0, ACC_ROWS), :] = xb[:, :DH]
    xb_ref[pl.ds(ACC_ROWS, ACC_ROWS), :] = xb[:, DH:]


def _make_tables(x_pad, w1at, w1bt):
    return pl.pallas_call(
        _tables_body,
        out_shape=(
            jax.ShapeDtypeStruct((2 * ACC_ROWS, DH), jnp.float32),
            jax.ShapeDtypeStruct((2 * ACC_ROWS, DH), jnp.float32),
        ),
    )(x_pad, w1at, w1bt)


_EB = 2560
_N_REAL_BLOCKS = N_EDGES // _EB


def _ea_body(attr_ref, wc_ref, b1_ref, ea_ref):
    ea_ref[...] = (
        jnp.dot(attr_ref[...], wc_ref[...], preferred_element_type=jnp.float32)
        + b1_ref[...]
    )


def _make_ea(edge_attr, w1ct, b1r):
    grid = (E_PAD // _EB,)
    return pl.pallas_call(
        _ea_body,
        grid=grid,
        in_specs=[
            pl.BlockSpec(
                (_EB, D_EDGE),
                lambda i: (jnp.minimum(i, _N_REAL_BLOCKS - 1), 0),
            ),
            pl.BlockSpec((D_EDGE, D), lambda i: (0, 0)),
            pl.BlockSpec((1, D), lambda i: (0, 0)),
        ],
        out_specs=pl.BlockSpec((_EB, D), lambda i: (i, 0)),
        out_shape=jax.ShapeDtypeStruct((E_PAD, D), jnp.float32),
    )(edge_attr, w1ct, b1r)


def _sc_body(
    xa_hbm, xb_hbm, ea_hbm, src_hbm, dst_hbm,
    acc_out, cnt_out,
    idx_dst0, idx_dst1,
    idx_srca0, idx_srca1, idx_dsta0, idx_dsta1,
    rows_a0, rows_a1, rows_b0, rows_b1, ea_buf0, ea_buf1,
    ones_buf, zero_cnt,
    acc_sh, cnt_sh,
    sem_a0, sem_a1, sem_b0, sem_b1, sem_e0, sem_e1,
):
    cid = lax.axis_index("c")
    sid = lax.axis_index("s")
    row_off = cid * ACC_ROWS
    col_off = cid * DH

    idx_dst = (idx_dst0, idx_dst1)
    idx_srca = (idx_srca0, idx_srca1)
    idx_dsta = (idx_dsta0, idx_dsta1)
    rows_a = (rows_a0, rows_a1)
    rows_b = (rows_b0, rows_b1)
    ea_buf = (ea_buf0, ea_buf1)
    sem_a = (sem_a0, sem_a1)
    sem_b = (sem_b0, sem_b1)
    sem_e = (sem_e0, sem_e1)

    def fill_rows(i, _):
        for j in range(DH // 16):
            rows_a0[i, pl.ds(j * 16, 16)] = jnp.zeros((16,), jnp.float32)
        ones_buf[i, pl.ds(0, 16)] = jnp.ones((16,), jnp.float32)
        zero_cnt[i, pl.ds(0, 16)] = jnp.zeros((16,), jnp.float32)
        return 0

    lax.fori_loop(0, C, fill_rows, 0)

    for k in range(ROWS_PT // C):
        pltpu.sync_copy(rows_a0, acc_sh.at[pl.ds(sid * ROWS_PT + k * C, C)])
        pltpu.sync_copy(zero_cnt, cnt_sh.at[pl.ds(sid * ROWS_PT + k * C, C)])
    plsc.subcore_barrier()

    ebase = sid * EPT

    def issue(k, b):
        base = ebase + k * C
        pltpu.sync_copy(src_hbm.at[pl.ds(base, C)], idx_srca[b])
        pltpu.sync_copy(dst_hbm.at[pl.ds(base, C)], idx_dst[b])
        for j in range(C // 16):
            s = pl.ds(j * 16, 16)
            idx_srca[b][s] = idx_srca[b][s] + row_off
            idx_dsta[b][s] = idx_dst[b][s] + row_off
        pltpu.async_copy(xa_hbm.at[idx_dsta[b]], rows_a[b], sem_a[b])
        pltpu.async_copy(xb_hbm.at[idx_srca[b]], rows_b[b], sem_b[b])
        pltpu.async_copy(
            ea_hbm.at[pl.ds(base, C), pl.ds(col_off, DH)], ea_buf[b], sem_e[b]
        )

    def consume(b):
        pltpu.make_async_copy(xa_hbm.at[idx_dsta[b]], rows_a[b], sem_a[b]).wait()
        pltpu.make_async_copy(xb_hbm.at[idx_srca[b]], rows_b[b], sem_b[b]).wait()
        pltpu.make_async_copy(
            ea_hbm.at[pl.ds(0, C), pl.ds(col_off, DH)], ea_buf[b], sem_e[b]
        ).wait()

        def row(i, _):
            for j in range(DH // 16):
                s = pl.ds(j * 16, 16)
                v = rows_a[b][i, s] + rows_b[b][i, s] + ea_buf[b][i, s]
                rows_a[b][i, s] = jnp.maximum(v, 0.0)
            return 0

        lax.fori_loop(0, C, row, 0)

        pltpu.sync_copy(rows_a[b], acc_sh.at[idx_dst[b]], add=True)

        @pl.when(cid == 0)
        def _():
            pltpu.sync_copy(ones_buf, cnt_sh.at[idx_dst[b]], add=True)

    issue(0, 0)

    def pair(k2, _):
        k = 2 * k2

        @pl.when(k + 1 < N_CHUNKS)
        def _():
            issue(k + 1, 1)

        consume(0)

        @pl.when(k + 2 < N_CHUNKS)
        def _():
            issue(k + 2, 0)

        consume(1)
        return 0

    lax.fori_loop(0, N_CHUNKS // 2, pair, 0)
    plsc.subcore_barrier()

    r0 = sid * ROWS_PT
    pltpu.sync_copy(acc_sh.at[pl.ds(r0, ROWS_PT)], acc_out.at[cid, pl.ds(r0, ROWS_PT)])

    @pl.when(cid == 0)
    def _():
        pltpu.sync_copy(cnt_sh.at[pl.ds(r0, ROWS_PT)], cnt_out.at[pl.ds(r0, ROWS_PT)])


def _make_sc(xa, xb, ea, src_p, dst_p):
    mesh = plsc.VectorSubcoreMesh(core_axis_name="c", subcore_axis_name="s")
    vm = pltpu.VMEM
    f32 = jnp.float32
    f = functools.partial(
        pl.kernel,
        compiler_params=pltpu.CompilerParams(use_tc_tiling_on_sc=False),
        out_type=(
            jax.ShapeDtypeStruct((2, ACC_ROWS, DH), f32),
            jax.ShapeDtypeStruct((ACC_ROWS, 16), f32),
        ),
        mesh=mesh,
        scratch_types=[
            vm((C,), jnp.int32), vm((C,), jnp.int32),
            vm((C,), jnp.int32), vm((C,), jnp.int32),
            vm((C,), jnp.int32), vm((C,), jnp.int32),
            vm((C, DH), f32), vm((C, DH), f32),
            vm((C, DH), f32), vm((C, DH), f32),
            vm((C, DH), f32), vm((C, DH), f32),
            vm((C, 16), f32),
            vm((C, 16), f32),
            pltpu.VMEM_SHARED((ACC_ROWS, DH), f32),
            pltpu.VMEM_SHARED((ACC_ROWS, 16), f32),
            pltpu.SemaphoreType.DMA, pltpu.SemaphoreType.DMA,
            pltpu.SemaphoreType.DMA, pltpu.SemaphoreType.DMA,
            pltpu.SemaphoreType.DMA, pltpu.SemaphoreType.DMA,
        ],
    )(_sc_body)
    return f(xa, xb, ea, src_p, dst_p)


def _finish_body(acc_ref, cnt_ref, w2t_ref, b2_ref, out_ref):
    s = jnp.concatenate(
        [acc_ref[0, :N_NODES, :], acc_ref[1, :N_NODES, :]], axis=1
    )
    c = jnp.maximum(cnt_ref[:N_NODES, 0:1], 1.0)
    out_ref[...] = (
        jnp.dot(s / c, w2t_ref[...], preferred_element_type=jnp.float32)
        + b2_ref[...]
    )


def _make_finish(acc, cnt, w2t, b2r):
    return pl.pallas_call(
        _finish_body,
        out_shape=jax.ShapeDtypeStruct((N_NODES, D), jnp.float32),
    )(acc, cnt, w2t, b2r)


@jax.jit
def kernel(x, edge_index, edge_attr, W1, b1, W2, b2):
    src = edge_index[0].astype(jnp.int32)
    dst = edge_index[1].astype(jnp.int32)

    n_pad = E_PAD - N_EDGES
    pad_idx = (N_NODES + jnp.arange(n_pad, dtype=jnp.int32) % N_PAD_ROWS)
    src_p = jnp.concatenate([src, pad_idx])
    dst_p = jnp.concatenate([dst, pad_idx])
    x_pad = jnp.concatenate(
        [x, jnp.zeros((ACC_ROWS - N_NODES, x.shape[1]), jnp.float32)]
    )

    w1at = W1[:, :D].T
    w1bt = W1[:, D:2 * D].T
    w1ct = W1[:, 2 * D:].T
    b1r = b1.reshape(1, D)
    b2r = b2.reshape(1, D)

    xa, xb = _make_tables(x_pad, w1at, w1bt)
    ea = _make_ea(edge_attr, w1ct, b1r)
    acc, cnt = _make_sc(xa, xb, ea, src_p, dst_p)
    return _make_finish(acc, cnt, W2.T, b2r)
